# Initial kernel scaffold; baseline (speedup 1.0000x reference)
import jax
import jax.numpy as jnp
from jax.experimental import pallas as pl


def kernel(pos, batch, tooth_n, emb_table, conv_emb_W, conv_emb_b, W1a, b1a, W1b, b1b, W2a, b2a, W2b, b2b, W3a, b3a, W3b, b3b, W4a, b4a, W4b, b4b, enc_W, enc_b, dec_W1, dec_b1, dec_W2, dec_b2):
    raise NotImplementedError("write your pallas kernel here")


# speedup vs baseline: 3.0033x; 3.0033x over previous
import functools
import jax
import jax.numpy as jnp
from jax.experimental import pallas as pl

K = 20
B = 8
NPTS = 2048
R = 256
BIG = 3.0e38


def _edge_conv_body(x_full_ref, x_row_ref, Wd_ref, Wn_ref, ba_ref, Wb_ref,
                    bb_ref, out_ref):
    xall = x_full_ref[0]
    xr = x_row_ref[0]
    f32 = jnp.float32
    hi = jax.lax.Precision.HIGHEST

    sqall = jnp.sum(xall * xall, axis=1)
    sqr = jnp.sum(xr * xr, axis=1)
    dot = jax.lax.dot_general(xr, xall, (((1,), (1,)), ((), ())),
                              preferred_element_type=f32, precision=hi)
    d = sqr[:, None] + sqall[None, :] - 2.0 * dot

    a = jax.lax.dot_general(xr, Wd_ref[...], (((1,), (0,)), ((), ())),
                            preferred_element_type=f32, precision=hi)
    a = a + ba_ref[...]
    y = jax.lax.dot_general(xall, Wn_ref[...], (((1,), (0,)), ((), ())),
                            preferred_element_type=f32, precision=hi)

    Wb = Wb_ref[...]
    iota = jax.lax.broadcasted_iota(jnp.int32, (R, NPTS), 1)
    acc = jnp.full((R, Wb.shape[1]), -BIG, dtype=f32)
    for _ in range(K):
        m = jnp.min(d, axis=1)
        cand = jnp.where(d <= m[:, None], iota, NPTS)
        idx = jnp.min(cand, axis=1)
        oh = iota == idx[:, None]
        d = jnp.where(oh, BIG, d)
        g = jax.lax.dot_general(oh.astype(f32), y, (((1,), (0,)), ((), ())),
                                preferred_element_type=f32, precision=hi)
        h = jax.lax.dot_general(jnp.maximum(a + g, 0.0), Wb,
                                (((1,), (0,)), ((), ())),
                                preferred_element_type=f32, precision=hi)
        acc = jnp.maximum(acc, h)
    out_ref[0] = acc + bb_ref[...]


def _edge_conv(x, Wd, Wn, ba, Wb, bb):
    C = x.shape[-1]
    H = Wd.shape[1]
    F = Wb.shape[1]
    grid = (B, NPTS // R)
    return pl.pallas_call(
        _edge_conv_body,
        grid=grid,
        in_specs=[
            pl.BlockSpec((1, NPTS, C), lambda b, r: (b, 0, 0)),
            pl.BlockSpec((1, R, C), lambda b, r: (b, r, 0)),
            pl.BlockSpec((C, H), lambda b, r: (0, 0)),
            pl.BlockSpec((C, H), lambda b, r: (0, 0)),
            pl.BlockSpec((1, H), lambda b, r: (0, 0)),
            pl.BlockSpec((H, F), lambda b, r: (0, 0)),
            pl.BlockSpec((1, F), lambda b, r: (0, 0)),
        ],
        out_specs=pl.BlockSpec((1, R, F), lambda b, r: (b, r, 0)),
        out_shape=jax.ShapeDtypeStruct((B, NPTS, F), jnp.float32),
    )(x, x, Wd, Wn, ba, Wb, bb)


def _maxpool_body(x1_ref, x2_ref, x3_ref, x4_ref, out_ref):
    m1 = jnp.max(x1_ref[0], axis=0)
    m2 = jnp.max(x2_ref[0], axis=0)
    m3 = jnp.max(x3_ref[0], axis=0)
    m4 = jnp.max(x4_ref[0], axis=0)
    out_ref[...] = jnp.concatenate([m1, m2, m3, m4], axis=0)[None, None, :]


def _maxpool(x1, x2, x3, x4):
    return pl.pallas_call(
        _maxpool_body,
        grid=(B,),
        in_specs=[
            pl.BlockSpec((1, NPTS, 64), lambda b: (b, 0, 0)),
            pl.BlockSpec((1, NPTS, 64), lambda b: (b, 0, 0)),
            pl.BlockSpec((1, NPTS, 64), lambda b: (b, 0, 0)),
            pl.BlockSpec((1, NPTS, 128), lambda b: (b, 0, 0)),
        ],
        out_specs=pl.BlockSpec((1, 1, 320), lambda b: (b, 0, 0)),
        out_shape=jax.ShapeDtypeStruct((B, 1, 320), jnp.float32),
    )(x1, x2, x3, x4).reshape(B, 320)


def _decoder_body(pooled_ref, tooth_ref, emb_table_ref, cembT_ref, cemb_b_ref,
                  encWa_ref, encWb_ref, enc_b_ref, dW1_ref, db1_ref,
                  dW2_ref, db2_ref, out_ref):
    f32 = jnp.float32
    hi = jax.lax.Precision.HIGHEST

    def mm(u, v):
        return jax.lax.dot_general(u, v, (((1,), (0,)), ((), ())),
                                   preferred_element_type=f32, precision=hi)

    tooth = tooth_ref[...]
    oh = (jax.lax.broadcasted_iota(jnp.int32, (B, 33), 1)
          == tooth).astype(f32)
    emb = mm(oh, emb_table_ref[...])
    emb = mm(emb, cembT_ref[...]) + cemb_b_ref[...]
    h = mm(pooled_ref[...], encWa_ref[...]) + mm(emb, encWb_ref[...])
    h = jnp.maximum(h + enc_b_ref[...], 0.0)
    h = jnp.maximum(mm(h, dW1_ref[...]) + db1_ref[...], 0.0)
    out_ref[...] = mm(h, dW2_ref[...]) + db2_ref[...]


def _decoder(pooled, tooth_n, emb_table, conv_emb_W, conv_emb_b,
             enc_W, enc_b, dec_W1, dec_b1, dec_W2, dec_b2):
    return pl.pallas_call(
        _decoder_body,
        out_shape=jax.ShapeDtypeStruct((B, 3072), jnp.float32),
    )(pooled, tooth_n.reshape(B, 1), emb_table, conv_emb_W.T,
      conv_emb_b.reshape(1, 64), enc_W[:320], enc_W[320:],
      enc_b.reshape(1, 512), dec_W1, dec_b1.reshape(1, 1024),
      dec_W2, dec_b2.reshape(1, 3072))


def kernel(pos, batch, tooth_n, emb_table, conv_emb_W, conv_emb_b,
           W1a, b1a, W1b, b1b, W2a, b2a, W2b, b2b,
           W3a, b3a, W3b, b3b, W4a, b4a, W4b, b4b,
           enc_W, enc_b, dec_W1, dec_b1, dec_W2, dec_b2):
    x0 = pos.reshape(B, NPTS, 3)
    x0 = jnp.concatenate([x0, jnp.zeros((B, NPTS, 5), jnp.float32)], axis=-1)

    def split(Wa, cpad=None):
        C = Wa.shape[0] // 2
        top, bot = Wa[:C], Wa[C:]
        Wd, Wn = top - bot, bot
        if cpad is not None and cpad > C:
            z = jnp.zeros((cpad - C, Wa.shape[1]), jnp.float32)
            Wd = jnp.concatenate([Wd, z], axis=0)
            Wn = jnp.concatenate([Wn, z], axis=0)
        return Wd, Wn

    Wd1, Wn1 = split(W1a, 8)
    x1 = _edge_conv(x0, Wd1, Wn1, b1a.reshape(1, -1), W1b, b1b.reshape(1, -1))
    Wd2, Wn2 = split(W2a)
    x2 = _edge_conv(x1, Wd2, Wn2, b2a.reshape(1, -1), W2b, b2b.reshape(1, -1))
    Wd3, Wn3 = split(W3a)
    x3 = _edge_conv(x2, Wd3, Wn3, b3a.reshape(1, -1), W3b, b3b.reshape(1, -1))
    Wd4, Wn4 = split(W4a)
    x4 = _edge_conv(x3, Wd4, Wn4, b4a.reshape(1, -1), W4b, b4b.reshape(1, -1))

    pooled = _maxpool(x1, x2, x3, x4)
    out = _decoder(pooled, tooth_n, emb_table, conv_emb_W, conv_emb_b,
                   enc_W, enc_b, dec_W1, dec_b1, dec_W2, dec_b2)
    return out.reshape(B, 1024, 3)


# speedup vs baseline: 6.8214x; 2.2713x over previous
import functools
import jax
import jax.numpy as jnp
from jax.experimental import pallas as pl

K = 20
B = 8
NPTS = 2048
R = 256
BIG = 3.0e38


def _edge_conv_body(x_full_ref, x_row_ref, Wd_ref, Wn_ref, ba_ref, Wb_ref,
                    bb_ref, out_ref):
    xall = x_full_ref[0]
    xr = x_row_ref[0]
    f32 = jnp.float32
    hi = jax.lax.Precision.HIGHEST

    sqall = jnp.sum(xall * xall, axis=1)
    sqr = jnp.sum(xr * xr, axis=1)
    dot = jax.lax.dot_general(xr, xall, (((1,), (1,)), ((), ())),
                              preferred_element_type=f32, precision=hi)
    d = sqr[:, None] + sqall[None, :] - 2.0 * dot

    a = jax.lax.dot_general(xr, Wd_ref[...], (((1,), (0,)), ((), ())),
                            preferred_element_type=f32, precision=hi)
    a = a + ba_ref[...]
    y = jax.lax.dot_general(xall, Wn_ref[...], (((1,), (0,)), ((), ())),
                            preferred_element_type=f32, precision=hi)
    y_hi = y.astype(jnp.bfloat16)
    y_lo = (y - y_hi.astype(f32)).astype(jnp.bfloat16)

    Wb = Wb_ref[...]
    iota = jax.lax.broadcasted_iota(jnp.int32, (R, NPTS), 1)
    acc = jnp.full((R, Wb.shape[1]), -BIG, dtype=f32)
    for _ in range(K):
        m = jnp.min(d, axis=1)
        cand = jnp.where(d <= m[:, None], iota, NPTS)
        idx = jnp.min(cand, axis=1)
        oh = iota == idx[:, None]
        d = jnp.where(oh, BIG, d)
        oh16 = oh.astype(jnp.bfloat16)
        g = (jax.lax.dot_general(oh16, y_hi, (((1,), (0,)), ((), ())),
                                 preferred_element_type=f32)
             + jax.lax.dot_general(oh16, y_lo, (((1,), (0,)), ((), ())),
                                   preferred_element_type=f32))
        h = jax.lax.dot_general(jnp.maximum(a + g, 0.0), Wb,
                                (((1,), (0,)), ((), ())),
                                preferred_element_type=f32, precision=hi)
        acc = jnp.maximum(acc, h)
    out_ref[0] = acc + bb_ref[...]


def _edge_conv(x, Wd, Wn, ba, Wb, bb):
    C = x.shape[-1]
    H = Wd.shape[1]
    F = Wb.shape[1]
    grid = (B, NPTS // R)
    return pl.pallas_call(
        _edge_conv_body,
        grid=grid,
        in_specs=[
            pl.BlockSpec((1, NPTS, C), lambda b, r: (b, 0, 0)),
            pl.BlockSpec((1, R, C), lambda b, r: (b, r, 0)),
            pl.BlockSpec((C, H), lambda b, r: (0, 0)),
            pl.BlockSpec((C, H), lambda b, r: (0, 0)),
            pl.BlockSpec((1, H), lambda b, r: (0, 0)),
            pl.BlockSpec((H, F), lambda b, r: (0, 0)),
            pl.BlockSpec((1, F), lambda b, r: (0, 0)),
        ],
        out_specs=pl.BlockSpec((1, R, F), lambda b, r: (b, r, 0)),
        out_shape=jax.ShapeDtypeStruct((B, NPTS, F), jnp.float32),
    )(x, x, Wd, Wn, ba, Wb, bb)


def _maxpool_body(x1_ref, x2_ref, x3_ref, x4_ref, out_ref):
    m1 = jnp.max(x1_ref[0], axis=0)
    m2 = jnp.max(x2_ref[0], axis=0)
    m3 = jnp.max(x3_ref[0], axis=0)
    m4 = jnp.max(x4_ref[0], axis=0)
    out_ref[...] = jnp.concatenate([m1, m2, m3, m4], axis=0)[None, None, :]


def _maxpool(x1, x2, x3, x4):
    return pl.pallas_call(
        _maxpool_body,
        grid=(B,),
        in_specs=[
            pl.BlockSpec((1, NPTS, 64), lambda b: (b, 0, 0)),
            pl.BlockSpec((1, NPTS, 64), lambda b: (b, 0, 0)),
            pl.BlockSpec((1, NPTS, 64), lambda b: (b, 0, 0)),
            pl.BlockSpec((1, NPTS, 128), lambda b: (b, 0, 0)),
        ],
        out_specs=pl.BlockSpec((1, 1, 320), lambda b: (b, 0, 0)),
        out_shape=jax.ShapeDtypeStruct((B, 1, 320), jnp.float32),
    )(x1, x2, x3, x4).reshape(B, 320)


def _decoder_body(pooled_ref, tooth_ref, emb_table_ref, cembT_ref, cemb_b_ref,
                  encWa_ref, encWb_ref, enc_b_ref, dW1_ref, db1_ref,
                  dW2_ref, db2_ref, out_ref):
    f32 = jnp.float32
    hi = jax.lax.Precision.HIGHEST

    def mm(u, v):
        return jax.lax.dot_general(u, v, (((1,), (0,)), ((), ())),
                                   preferred_element_type=f32, precision=hi)

    tooth = tooth_ref[...]
    oh = (jax.lax.broadcasted_iota(jnp.int32, (B, 33), 1)
          == tooth).astype(f32)
    emb = mm(oh, emb_table_ref[...])
    emb = mm(emb, cembT_ref[...]) + cemb_b_ref[...]
    h = mm(pooled_ref[...], encWa_ref[...]) + mm(emb, encWb_ref[...])
    h = jnp.maximum(h + enc_b_ref[...], 0.0)
    h = jnp.maximum(mm(h, dW1_ref[...]) + db1_ref[...], 0.0)
    out_ref[...] = mm(h, dW2_ref[...]) + db2_ref[...]


def _decoder(pooled, tooth_n, emb_table, conv_emb_W, conv_emb_b,
             enc_W, enc_b, dec_W1, dec_b1, dec_W2, dec_b2):
    return pl.pallas_call(
        _decoder_body,
        out_shape=jax.ShapeDtypeStruct((B, 3072), jnp.float32),
    )(pooled, tooth_n.reshape(B, 1), emb_table, conv_emb_W.T,
      conv_emb_b.reshape(1, 64), enc_W[:320], enc_W[320:],
      enc_b.reshape(1, 512), dec_W1, dec_b1.reshape(1, 1024),
      dec_W2, dec_b2.reshape(1, 3072))


def kernel(pos, batch, tooth_n, emb_table, conv_emb_W, conv_emb_b,
           W1a, b1a, W1b, b1b, W2a, b2a, W2b, b2b,
           W3a, b3a, W3b, b3b, W4a, b4a, W4b, b4b,
           enc_W, enc_b, dec_W1, dec_b1, dec_W2, dec_b2):
    x0 = pos.reshape(B, NPTS, 3)
    x0 = jnp.concatenate([x0, jnp.zeros((B, NPTS, 5), jnp.float32)], axis=-1)

    def split(Wa, cpad=None):
        C = Wa.shape[0] // 2
        top, bot = Wa[:C], Wa[C:]
        Wd, Wn = top - bot, bot
        if cpad is not None and cpad > C:
            z = jnp.zeros((cpad - C, Wa.shape[1]), jnp.float32)
            Wd = jnp.concatenate([Wd, z], axis=0)
            Wn = jnp.concatenate([Wn, z], axis=0)
        return Wd, Wn

    Wd1, Wn1 = split(W1a, 8)
    x1 = _edge_conv(x0, Wd1, Wn1, b1a.reshape(1, -1), W1b, b1b.reshape(1, -1))
    Wd2, Wn2 = split(W2a)
    x2 = _edge_conv(x1, Wd2, Wn2, b2a.reshape(1, -1), W2b, b2b.reshape(1, -1))
    Wd3, Wn3 = split(W3a)
    x3 = _edge_conv(x2, Wd3, Wn3, b3a.reshape(1, -1), W3b, b3b.reshape(1, -1))
    Wd4, Wn4 = split(W4a)
    x4 = _edge_conv(x3, Wd4, Wn4, b4a.reshape(1, -1), W4b, b4b.reshape(1, -1))

    pooled = _maxpool(x1, x2, x3, x4)
    out = _decoder(pooled, tooth_n, emb_table, conv_emb_W, conv_emb_b,
                   enc_W, enc_b, dec_W1, dec_b1, dec_W2, dec_b2)
    return out.reshape(B, 1024, 3)
